# baseline (speedup 1.0000x reference)
import functools

import jax
import jax.numpy as jnp
from jax import lax
import jax.experimental.pallas as pl
from jax.experimental.pallas import tpu as pltpu
from jax.experimental.pallas import tpu_sc as plsc

N = 10000
D = 128
E = 320000
EPS_BN = 1e-5
EPS_L2 = 1e-12

NC = 2
NS = 16
NW = NC * NS
CH = 128
NCHP = 160
E_PAD = NS * NCHP * CH
N_ACC = 10112
RPT = N_ACC // NS
N_TAB = 10016
CFAST = 0
NCHF = 112
NCHS = 48
HFB = 56
NB = 2

_mesh = plsc.VectorSubcoreMesh(core_axis_name="c", subcore_axis_name="s")
_f32 = jnp.float32


DW = 128
NCHD = E_PAD // (NS * CH)


@functools.partial(
    pl.kernel,
    out_type=jax.ShapeDtypeStruct((NC, N_ACC, DW), _f32),
    mesh=_mesh,
    scratch_types=[
        pltpu.VMEM((NCHD, CH), jnp.int32),
        pltpu.VMEM((CH, DW), _f32),
        pltpu.VMEM_SHARED((N_ACC, DW), _f32),
    ],
)
def _deg_kernel(edges_hbm, ones_hbm, zr_hbm, out_hbm, idx_v, ones_v, acc):
  c = lax.axis_index("c")
  s = lax.axis_index("s")
  pltpu.sync_copy(edges_hbm.at[c, s], idx_v)
  pltpu.sync_copy(ones_hbm, ones_v)
  pltpu.sync_copy(zr_hbm, acc.at[pl.ds(s * RPT, RPT)])
  plsc.subcore_barrier()

  @pl.loop(0, NCHD)
  def _(j):
    pltpu.sync_copy(ones_v, acc.at[idx_v.at[j]], add=True)

  plsc.subcore_barrier()
  rows = pl.ds(s * RPT, RPT)
  pltpu.sync_copy(acc.at[rows], out_hbm.at[c, rows])


def _seg_span(tab_hbm, src_slab, dst_slab, cnt, src_v, dst_v, rows, acc,
              sgs, sss):
  pltpu.sync_copy(src_slab, src_v.at[pl.ds(0, cnt)])
  pltpu.sync_copy(dst_slab, dst_v.at[pl.ds(0, cnt)])

  for b in range(NB):
    pltpu.make_async_copy(tab_hbm.at[src_v.at[b]], rows[b], sgs[b]).start()

  @pl.loop(0, cnt, step=NB)
  def _(j):
    for b in range(NB):
      pltpu.make_async_copy(tab_hbm.at[src_v.at[j + b]], rows[b],
                            sgs[b]).wait()
      pltpu.make_async_copy(rows[b], acc.at[dst_v.at[j + b]],
                            sss[b]).start(add=True)
    for b in range(NB):
      pltpu.make_async_copy(rows[b], acc.at[dst_v.at[j + b]],
                            sss[b]).wait()

      @pl.when(j + b + NB < cnt)
      def _():
        pltpu.make_async_copy(tab_hbm.at[src_v.at[j + b + NB]], rows[b],
                              sgs[b]).start()


@functools.partial(
    pl.kernel,
    out_type=jax.ShapeDtypeStruct((NC, N_ACC, D), _f32),
    mesh=_mesh,
    scratch_types=(
        [pltpu.VMEM((HFB, CH), jnp.int32) for _ in range(2)]
        + [pltpu.VMEM((CH, D), _f32) for _ in range(NB)]
        + [pltpu.VMEM_SHARED((N_ACC, D), _f32)]
        + [pltpu.SemaphoreType.DMA for _ in range(2 * NB)]
    ),
)
def _segsum_kernel(tab_hbm, srcf_hbm, dstf_hbm, srcs_hbm, dsts_hbm, zb_hbm,
                   out_hbm, src_v, dst_v, *rest):
  rows = rest[:NB]
  acc = rest[NB]
  sgs = rest[NB + 1:2 * NB + 1]
  sss = rest[2 * NB + 1:]
  c = lax.axis_index("c")
  s = lax.axis_index("s")

  pltpu.sync_copy(zb_hbm, acc.at[pl.ds(s * RPT, RPT)])
  plsc.subcore_barrier()

  @pl.when(c == CFAST)
  def _():
    for hh in range(NCHF // HFB):
      sl = pl.ds(hh * HFB, HFB)
      _seg_span(tab_hbm, srcf_hbm.at[s, sl], dstf_hbm.at[s, sl], HFB,
                src_v, dst_v, rows, acc, sgs, sss)

  @pl.when(c != CFAST)
  def _():
    _seg_span(tab_hbm, srcs_hbm.at[s], dsts_hbm.at[s], NCHS,
              src_v, dst_v, rows, acc, sgs, sss)

  plsc.subcore_barrier()
  rows_sl = pl.ds(s * RPT, RPT)
  pltpu.sync_copy(acc.at[rows_sl], out_hbm.at[c, rows_sl])


def _norms_body(degp_ref, ns_ref, nd_ref):
  dp = degp_ref[...]
  ns_ref[...] = jnp.broadcast_to(
      lax.rsqrt(jnp.maximum(dp[0, :, 0:1], 1.0)), (N_ACC, D))
  nd_ref[...] = jnp.broadcast_to(
      lax.rsqrt(jnp.maximum(dp[1, :, 0:1], 1.0)), (N_ACC, D))


_norms = pl.pallas_call(
    _norms_body,
    out_shape=[jax.ShapeDtypeStruct((N_ACC, D), _f32)] * 2,
)


def _prep_body(h_ref, w_ref, b_ref, ns_ref, out_ref):
  t = jnp.dot(h_ref[...], w_ref[...], preferred_element_type=_f32)
  t = t + b_ref[...]
  out_ref[0:N, :] = t * ns_ref[0:N, :]
  out_ref[N:N_TAB, :] = jnp.zeros((N_TAB - N, D), _f32)


_prep = pl.pallas_call(
    _prep_body,
    out_shape=jax.ShapeDtypeStruct((N_TAB, D), _f32),
)


def _post(p_ref, nd_ref, g_ref, be_ref):
  x = (p_ref[0, 0:N, :] + p_ref[1, 0:N, :]) * nd_ref[0:N, :]
  mu = jnp.mean(x, axis=0, keepdims=True)
  xc = x - mu
  var = jnp.mean(xc * xc, axis=0, keepdims=True)
  y = xc * lax.rsqrt(var + EPS_BN) * g_ref[...] + be_ref[...]
  y = jnp.maximum(y, 0.0)
  rn = jnp.sqrt(jnp.sum(y * y, axis=1, keepdims=True))
  return y / jnp.maximum(rn, EPS_L2)


def _mid_body(p_ref, nd_ref, g_ref, be_ref, w_ref, b_ref, ns_ref, out_ref):
  y = _post(p_ref, nd_ref, g_ref, be_ref)
  t = jnp.dot(y, w_ref[...], preferred_element_type=_f32) + b_ref[...]
  out_ref[0:N, :] = t * ns_ref[0:N, :]
  out_ref[N:N_TAB, :] = jnp.zeros((N_TAB - N, D), _f32)


_mid = pl.pallas_call(
    _mid_body,
    out_shape=jax.ShapeDtypeStruct((N_TAB, D), _f32),
)


def _final_body(p_ref, nd_ref, g_ref, be_ref, out_ref):
  y = _post(p_ref, nd_ref, g_ref, be_ref)
  rn = jnp.sqrt(jnp.sum(y * y, axis=1, keepdims=True))
  out_ref[...] = y / jnp.maximum(rn, EPS_L2)


_final = pl.pallas_call(
    _final_body,
    out_shape=jax.ShapeDtypeStruct((N, D), _f32),
)


def kernel(edge_index, h, W0, b0, g0, be0, W1, b1, g1, be1):
  pad = E_PAD - E
  padv = jnp.full((pad,), N, jnp.int32)
  src_all = jnp.concatenate([edge_index[0], padv])
  dst_all = jnp.concatenate([edge_index[1], padv])
  srcp = src_all.reshape(NS, NCHP, CH)
  dstp = dst_all.reshape(NS, NCHP, CH)
  srcf, srcs = srcp[:, :NCHF], srcp[:, NCHF:]
  dstf, dsts = dstp[:, :NCHF], dstp[:, NCHF:]
  edges2 = jnp.stack([src_all, dst_all]).reshape(2, NS, NCHD, CH)
  onesd = jnp.ones((CH, DW), _f32)
  zrd = jnp.zeros((RPT, DW), _f32)
  zbig = jnp.zeros((RPT, D), _f32)

  degp = _deg_kernel(edges2, onesd, zrd)
  ns, nd = _norms(degp)

  t0 = _prep(h, W0, b0.reshape(1, D), ns)
  p0 = _segsum_kernel(t0, srcf, dstf, srcs, dsts, zbig)
  t1 = _mid(p0, nd, g0.reshape(1, D), be0.reshape(1, D),
            W1, b1.reshape(1, D), ns)
  p1 = _segsum_kernel(t1, srcf, dstf, srcs, dsts, zbig)
  return _final(p1, nd, g1.reshape(1, D), be1.reshape(1, D))


# speedup vs baseline: 1.0032x; 1.0032x over previous
import functools

import jax
import jax.numpy as jnp
from jax import lax
import jax.experimental.pallas as pl
from jax.experimental.pallas import tpu as pltpu
from jax.experimental.pallas import tpu_sc as plsc

N = 10000
D = 128
E = 320000
EPS_BN = 1e-5
EPS_L2 = 1e-12

NC = 2
NS = 16
NW = NC * NS
CH = 128
NCHP = 160
E_PAD = NS * NCHP * CH
N_ACC = 10112
RPT = N_ACC // NS
N_TAB = 10016
CFAST = 0
NCHF = 120
NCHS = 40
HFB = 40
NB = 2

_mesh = plsc.VectorSubcoreMesh(core_axis_name="c", subcore_axis_name="s")
_f32 = jnp.float32


DW = 128
NCHD = E_PAD // (NS * CH)


@functools.partial(
    pl.kernel,
    out_type=jax.ShapeDtypeStruct((NC, N_ACC, DW), _f32),
    mesh=_mesh,
    scratch_types=[
        pltpu.VMEM((NCHD, CH), jnp.int32),
        pltpu.VMEM((CH, DW), _f32),
        pltpu.VMEM_SHARED((N_ACC, DW), _f32),
        pltpu.SemaphoreType.DMA,
    ],
)
def _deg_kernel(edges_hbm, ones_hbm, zr_hbm, out_hbm, idx_v, ones_v, acc,
                sem):
  c = lax.axis_index("c")
  s = lax.axis_index("s")
  pltpu.sync_copy(edges_hbm.at[c, s], idx_v)
  pltpu.sync_copy(ones_hbm, ones_v)
  pltpu.sync_copy(zr_hbm, acc.at[pl.ds(s * RPT, RPT)])
  plsc.subcore_barrier()

  @pl.loop(0, NCHD)
  def _(j):
    pltpu.make_async_copy(ones_v, acc.at[idx_v.at[j]], sem).start(add=True)

    @pl.when(j >= 8)
    def _():
      pltpu.make_async_copy(ones_v, acc.at[idx_v.at[j - 8]], sem).wait()

  @pl.loop(NCHD - 8, NCHD)
  def _(j):
    pltpu.make_async_copy(ones_v, acc.at[idx_v.at[j]], sem).wait()

  plsc.subcore_barrier()
  rows = pl.ds(s * RPT, RPT)
  pltpu.sync_copy(acc.at[rows], out_hbm.at[c, rows])


def _seg_span(tab_hbm, src_slab, dst_slab, cnt, src_v, dst_v, rows, acc,
              sgs, sss):
  pltpu.sync_copy(src_slab, src_v.at[pl.ds(0, cnt)])
  pltpu.sync_copy(dst_slab, dst_v.at[pl.ds(0, cnt)])

  for b in range(NB):
    pltpu.make_async_copy(tab_hbm.at[src_v.at[b]], rows[b], sgs[b]).start()

  @pl.loop(0, cnt, step=NB)
  def _(j):
    for b in range(NB):
      pltpu.make_async_copy(tab_hbm.at[src_v.at[j + b]], rows[b],
                            sgs[b]).wait()
      pltpu.make_async_copy(rows[b], acc.at[dst_v.at[j + b]],
                            sss[b]).start(add=True)
    for b in range(NB):
      pltpu.make_async_copy(rows[b], acc.at[dst_v.at[j + b]],
                            sss[b]).wait()

      @pl.when(j + b + NB < cnt)
      def _():
        pltpu.make_async_copy(tab_hbm.at[src_v.at[j + b + NB]], rows[b],
                              sgs[b]).start()


@functools.partial(
    pl.kernel,
    out_type=jax.ShapeDtypeStruct((NC, N_ACC, D), _f32),
    mesh=_mesh,
    scratch_types=(
        [pltpu.VMEM((HFB, CH), jnp.int32) for _ in range(2)]
        + [pltpu.VMEM((CH, D), _f32) for _ in range(NB)]
        + [pltpu.VMEM_SHARED((N_ACC, D), _f32)]
        + [pltpu.SemaphoreType.DMA for _ in range(2 * NB)]
    ),
)
def _segsum_kernel(tab_hbm, srcf_hbm, dstf_hbm, srcs_hbm, dsts_hbm, zb_hbm,
                   out_hbm, src_v, dst_v, *rest):
  rows = rest[:NB]
  acc = rest[NB]
  sgs = rest[NB + 1:2 * NB + 1]
  sss = rest[2 * NB + 1:]
  c = lax.axis_index("c")
  s = lax.axis_index("s")

  pltpu.sync_copy(zb_hbm, acc.at[pl.ds(s * RPT, RPT)])
  plsc.subcore_barrier()

  @pl.when(c == CFAST)
  def _():
    for hh in range(NCHF // HFB):
      sl = pl.ds(hh * HFB, HFB)
      _seg_span(tab_hbm, srcf_hbm.at[s, sl], dstf_hbm.at[s, sl], HFB,
                src_v, dst_v, rows, acc, sgs, sss)

  @pl.when(c != CFAST)
  def _():
    _seg_span(tab_hbm, srcs_hbm.at[s], dsts_hbm.at[s], NCHS,
              src_v, dst_v, rows, acc, sgs, sss)

  plsc.subcore_barrier()
  rows_sl = pl.ds(s * RPT, RPT)
  pltpu.sync_copy(acc.at[rows_sl], out_hbm.at[c, rows_sl])


def _norms_body(degp_ref, ns_ref, nd_ref):
  dp = degp_ref[...]
  ns_ref[...] = jnp.broadcast_to(
      lax.rsqrt(jnp.maximum(dp[0, :, 0:1], 1.0)), (N_ACC, D))
  nd_ref[...] = jnp.broadcast_to(
      lax.rsqrt(jnp.maximum(dp[1, :, 0:1], 1.0)), (N_ACC, D))


_norms = pl.pallas_call(
    _norms_body,
    out_shape=[jax.ShapeDtypeStruct((N_ACC, D), _f32)] * 2,
)


def _prep_body(h_ref, w_ref, b_ref, ns_ref, out_ref):
  t = jnp.dot(h_ref[...], w_ref[...], preferred_element_type=_f32)
  t = t + b_ref[...]
  out_ref[0:N, :] = t * ns_ref[0:N, :]
  out_ref[N:N_TAB, :] = jnp.zeros((N_TAB - N, D), _f32)


_prep = pl.pallas_call(
    _prep_body,
    out_shape=jax.ShapeDtypeStruct((N_TAB, D), _f32),
)


def _post(p_ref, nd_ref, g_ref, be_ref):
  x = (p_ref[0, 0:N, :] + p_ref[1, 0:N, :]) * nd_ref[0:N, :]
  mu = jnp.mean(x, axis=0, keepdims=True)
  xc = x - mu
  var = jnp.mean(xc * xc, axis=0, keepdims=True)
  y = xc * lax.rsqrt(var + EPS_BN) * g_ref[...] + be_ref[...]
  y = jnp.maximum(y, 0.0)
  rn = jnp.sqrt(jnp.sum(y * y, axis=1, keepdims=True))
  return y / jnp.maximum(rn, EPS_L2)


def _mid_body(p_ref, nd_ref, g_ref, be_ref, w_ref, b_ref, ns_ref, out_ref):
  y = _post(p_ref, nd_ref, g_ref, be_ref)
  t = jnp.dot(y, w_ref[...], preferred_element_type=_f32) + b_ref[...]
  out_ref[0:N, :] = t * ns_ref[0:N, :]
  out_ref[N:N_TAB, :] = jnp.zeros((N_TAB - N, D), _f32)


_mid = pl.pallas_call(
    _mid_body,
    out_shape=jax.ShapeDtypeStruct((N_TAB, D), _f32),
)


def _final_body(p_ref, nd_ref, g_ref, be_ref, out_ref):
  y = _post(p_ref, nd_ref, g_ref, be_ref)
  rn = jnp.sqrt(jnp.sum(y * y, axis=1, keepdims=True))
  out_ref[...] = y / jnp.maximum(rn, EPS_L2)


_final = pl.pallas_call(
    _final_body,
    out_shape=jax.ShapeDtypeStruct((N, D), _f32),
)


def kernel(edge_index, h, W0, b0, g0, be0, W1, b1, g1, be1):
  pad = E_PAD - E
  padv = jnp.full((pad,), N, jnp.int32)
  src_all = jnp.concatenate([edge_index[0], padv])
  dst_all = jnp.concatenate([edge_index[1], padv])
  srcp = src_all.reshape(NS, NCHP, CH)
  dstp = dst_all.reshape(NS, NCHP, CH)
  srcf, srcs = srcp[:, :NCHF], srcp[:, NCHF:]
  dstf, dsts = dstp[:, :NCHF], dstp[:, NCHF:]
  edges2 = jnp.stack([src_all, dst_all]).reshape(2, NS, NCHD, CH)
  onesd = jnp.ones((CH, DW), _f32)
  zrd = jnp.zeros((RPT, DW), _f32)
  zbig = jnp.zeros((RPT, D), _f32)

  degp = _deg_kernel(edges2, onesd, zrd)
  ns, nd = _norms(degp)

  t0 = _prep(h, W0, b0.reshape(1, D), ns)
  p0 = _segsum_kernel(t0, srcf, dstf, srcs, dsts, zbig)
  t1 = _mid(p0, nd, g0.reshape(1, D), be0.reshape(1, D),
            W1, b1.reshape(1, D), ns)
  p1 = _segsum_kernel(t1, srcf, dstf, srcs, dsts, zbig)
  return _final(p1, nd, g1.reshape(1, D), be1.reshape(1, D))
